# Initial kernel scaffold; baseline (speedup 1.0000x reference)
import jax
import jax.numpy as jnp
from jax.experimental import pallas as pl


def kernel(x, edge_index, W1, b1, Wlin, blin, W2, b2, Wout, bout):
    raise NotImplementedError("write your pallas kernel here")


# speedup vs baseline: 10.4321x; 10.4321x over previous
import functools

import jax
import jax.numpy as jnp
from jax import lax
from jax.experimental import pallas as pl
from jax.experimental.pallas import tpu as pltpu
from jax.experimental.pallas import tpu_sc as plsc

N_NODES = 10000
D = 128
OUTD = 64
E = 320000

NPAD = 10240
CHUNK = 128
NW = 32
CPW = 79
NCHUNKS = NW * CPW
EPAD = NCHUNKS * CHUNK
RPW = NPAD // 16
BLK = 1024


@functools.cache
def _get_deg_kernel():
    mesh = plsc.VectorSubcoreMesh(core_axis_name="c", subcore_axis_name="s")

    @functools.partial(
        pl.kernel,
        out_type=jax.ShapeDtypeStruct((2, NPAD, D), jnp.float32),
        mesh=mesh,
        scratch_types=[
            pltpu.VMEM((1, CHUNK), jnp.int32),
            pltpu.VMEM((CHUNK, D), jnp.float32),
            pltpu.VMEM_SHARED((NPAD, D), jnp.float32),
        ],
    )
    def _deg_kernel(dst_hbm, ones_hbm, zeros_hbm, out_hbm, idx_v, ones_v,
                    acc_sh):
        cid = lax.axis_index("c")
        sid = lax.axis_index("s")
        pltpu.sync_copy(zeros_hbm, acc_sh.at[pl.ds(sid * RPW, RPW)])
        pltpu.sync_copy(ones_hbm, ones_v)
        plsc.subcore_barrier()
        w = cid * 16 + sid

        @pl.loop(0, CPW)
        def _(i):
            k = w * CPW + i
            pltpu.sync_copy(dst_hbm.at[pl.ds(k, 1)], idx_v)
            pltpu.sync_copy(ones_v, acc_sh.at[idx_v.at[0]], add=True)

        plsc.subcore_barrier()
        pltpu.sync_copy(acc_sh.at[pl.ds(sid * RPW, RPW)],
                        out_hbm.at[cid, pl.ds(sid * RPW, RPW)])

    return _deg_kernel


@functools.cache
def _get_scatter_kernel():
    mesh = plsc.VectorSubcoreMesh(core_axis_name="c", subcore_axis_name="s")

    @functools.partial(
        pl.kernel,
        out_type=jax.ShapeDtypeStruct((2, NPAD, D), jnp.float32),
        mesh=mesh,
        scratch_types=[
            pltpu.VMEM((1, CHUNK), jnp.int32),
            pltpu.VMEM((1, CHUNK), jnp.int32),
            pltpu.VMEM((CHUNK, D), jnp.float32),
            pltpu.VMEM_SHARED((NPAD, D), jnp.float32),
        ],
    )
    def _scatter_kernel(g_hbm, src_hbm, dst_hbm, zeros_hbm, out_hbm,
                        idxs_v, idxd_v, rows_v, acc_sh):
        cid = lax.axis_index("c")
        sid = lax.axis_index("s")
        pltpu.sync_copy(zeros_hbm, acc_sh.at[pl.ds(sid * RPW, RPW)])
        plsc.subcore_barrier()
        w = cid * 16 + sid

        @pl.loop(0, CPW)
        def _(i):
            k = w * CPW + i
            pltpu.sync_copy(src_hbm.at[pl.ds(k, 1)], idxs_v)
            pltpu.sync_copy(dst_hbm.at[pl.ds(k, 1)], idxd_v)
            pltpu.sync_copy(g_hbm.at[idxs_v.at[0]], rows_v)
            pltpu.sync_copy(rows_v, acc_sh.at[idxd_v.at[0]], add=True)

        plsc.subcore_barrier()
        pltpu.sync_copy(acc_sh.at[pl.ds(sid * RPW, RPW)],
                        out_hbm.at[cid, pl.ds(sid * RPW, RPW)])

    return _scatter_kernel


def _mm_body(x_ref, w_ref, o_ref):
    o_ref[...] = jnp.dot(x_ref[...], w_ref[...],
                         preferred_element_type=jnp.float32)


def _tc_matmul(x, w):
    n, k = x.shape
    m = w.shape[1]
    return pl.pallas_call(
        _mm_body,
        grid=(n // BLK,),
        in_specs=[pl.BlockSpec((BLK, k), lambda i: (i, 0)),
                  pl.BlockSpec((k, m), lambda i: (0, 0))],
        out_specs=pl.BlockSpec((BLK, m), lambda i: (i, 0)),
        out_shape=jax.ShapeDtypeStruct((n, m), jnp.float32),
    )(x, w)


def _prep_body(degp_ref, h1_ref, dinv_ref, g1_ref):
    deg = degp_ref[0] + degp_ref[1] + 1.0
    dinv = lax.rsqrt(deg)
    dinv_ref[...] = dinv
    g1_ref[...] = h1_ref[...] * dinv


def _tc_prep(degp, h1):
    return pl.pallas_call(
        _prep_body,
        grid=(NPAD // BLK,),
        in_specs=[pl.BlockSpec((2, BLK, D), lambda i: (0, i, 0)),
                  pl.BlockSpec((BLK, D), lambda i: (i, 0))],
        out_specs=[pl.BlockSpec((BLK, D), lambda i: (i, 0)),
                   pl.BlockSpec((BLK, D), lambda i: (i, 0))],
        out_shape=[jax.ShapeDtypeStruct((NPAD, D), jnp.float32),
                   jax.ShapeDtypeStruct((NPAD, D), jnp.float32)],
    )(degp, h1)


def _mid_body(s_ref, g1_ref, dinv_ref, b1_ref, wlin_ref, blin_ref, w2_ref,
              g2_ref):
    dinv = dinv_ref[...]
    s = s_ref[0] + s_ref[1] + g1_ref[...]
    a = s * dinv + b1_ref[...]
    t = jnp.tanh(a)
    l = jnp.dot(t, wlin_ref[...], preferred_element_type=jnp.float32)
    l = l + blin_ref[...]
    h2 = jnp.dot(l, w2_ref[...], preferred_element_type=jnp.float32)
    g2_ref[...] = h2 * dinv


def _tc_mid(s1, g1, dinv16, b1r, Wlin, blinr, W2):
    return pl.pallas_call(
        _mid_body,
        grid=(NPAD // BLK,),
        in_specs=[pl.BlockSpec((2, BLK, D), lambda i: (0, i, 0)),
                  pl.BlockSpec((BLK, D), lambda i: (i, 0)),
                  pl.BlockSpec((BLK, D), lambda i: (i, 0)),
                  pl.BlockSpec((1, D), lambda i: (0, 0)),
                  pl.BlockSpec((D, D), lambda i: (0, 0)),
                  pl.BlockSpec((1, D), lambda i: (0, 0)),
                  pl.BlockSpec((D, D), lambda i: (0, 0))],
        out_specs=pl.BlockSpec((BLK, D), lambda i: (i, 0)),
        out_shape=jax.ShapeDtypeStruct((NPAD, D), jnp.float32),
    )(s1, g1, dinv16, b1r, Wlin, blinr, W2)


def _fin_body(s_ref, g2_ref, dinv_ref, b2_ref, wout_ref, bout_ref, o_ref):
    i = pl.program_id(0)
    dinv = dinv_ref[...]
    a = (s_ref[0] + s_ref[1] + g2_ref[...]) * dinv + b2_ref[...]
    t = jnp.tanh(a)
    o = jnp.dot(t, wout_ref[...], preferred_element_type=jnp.float32)
    o = o + bout_ref[...]
    rows = lax.broadcasted_iota(jnp.int32, (BLK, OUTD), 0) + i * BLK
    o = jnp.where(rows < N_NODES, o, -jnp.inf)
    m = jnp.max(o, axis=0, keepdims=True)

    @pl.when(i == 0)
    def _():
        o_ref[...] = m

    @pl.when(i != 0)
    def _():
        o_ref[...] = jnp.maximum(o_ref[...], m)


def _tc_final(s2, g2, dinv16, b2r, Wout, boutr):
    return pl.pallas_call(
        _fin_body,
        grid=(NPAD // BLK,),
        in_specs=[pl.BlockSpec((2, BLK, D), lambda i: (0, i, 0)),
                  pl.BlockSpec((BLK, D), lambda i: (i, 0)),
                  pl.BlockSpec((BLK, D), lambda i: (i, 0)),
                  pl.BlockSpec((1, D), lambda i: (0, 0)),
                  pl.BlockSpec((D, OUTD), lambda i: (0, 0)),
                  pl.BlockSpec((1, OUTD), lambda i: (0, 0))],
        out_specs=pl.BlockSpec((1, OUTD), lambda i: (0, 0)),
        out_shape=jax.ShapeDtypeStruct((1, OUTD), jnp.float32),
    )(s2, g2, dinv16, b2r, Wout, boutr)


def kernel(x, edge_index, W1, b1, Wlin, blin, W2, b2, Wout, bout):
    src = edge_index[0].astype(jnp.int32)
    dst = edge_index[1].astype(jnp.int32)
    src_c = jnp.pad(src, (0, EPAD - E)).reshape(NCHUNKS, CHUNK)
    dst_c = jnp.pad(dst, (0, EPAD - E),
                    constant_values=N_NODES).reshape(NCHUNKS, CHUNK)
    x_p = jnp.pad(x, ((0, NPAD - N_NODES), (0, 0)))
    zerosD = jnp.zeros((RPW, D), jnp.float32)
    onesD = jnp.ones((CHUNK, D), jnp.float32)
    b1r = b1.reshape(1, D)
    blinr = blin.reshape(1, D)
    b2r = b2.reshape(1, D)
    boutr = bout.reshape(1, OUTD)

    deg_kernel = _get_deg_kernel()
    scatter_kernel = _get_scatter_kernel()
    degp = deg_kernel(dst_c, onesD, zerosD)
    h1 = _tc_matmul(x_p, W1)
    dinv, g1 = _tc_prep(degp, h1)
    s1 = scatter_kernel(g1, src_c, dst_c, zerosD)
    g2 = _tc_mid(s1, g1, dinv, b1r, Wlin, blinr, W2)
    s2 = scatter_kernel(g2, src_c, dst_c, zerosD)
    return _tc_final(s2, g2, dinv, b2r, Wout, boutr)
